# Initial kernel scaffold; baseline (speedup 1.0000x reference)
import jax
import jax.numpy as jnp
from jax.experimental import pallas as pl


def kernel(src, in_proj_w, in_proj_b, out_w, out_b, norm1_g, norm1_b, gate_w, gate_b, W1, b1, W2, b2, norm2_g, norm2_b):
    raise NotImplementedError("write your pallas kernel here")


# speedup vs baseline: 1.5076x; 1.5076x over previous
import functools

import jax
import jax.numpy as jnp
import numpy as np
from jax import lax
from jax.experimental import pallas as pl
from jax.experimental.pallas import tpu as pltpu

S, D, H, DFF, E, K = 2048, 768, 12, 3072, 8, 2
DH = D // H
TQ = 256
NTB = S // TQ
T = 128
PAD = S * K + E * T
NBLK = PAD // T
EPAD = 128
NEG = -1e30


def _qkv_body(x_ref, w_ref, b_ref, o_ref):
    acc = jnp.dot(x_ref[...], w_ref[...], preferred_element_type=jnp.float32)
    o_ref[...] = (acc + b_ref[...]).astype(jnp.bfloat16)


def _attn_body(q_ref, k_ref, v_ref, o_ref):
    q = q_ref[0]
    k = k_ref[0]
    v = v_ref[0]
    s = lax.dot_general(q, k, (((1,), (1,)), ((), ())),
                        preferred_element_type=jnp.float32)
    s = s * (1.0 / np.sqrt(DH).astype(np.float32))
    m = jnp.max(s, axis=-1, keepdims=True)
    p = jnp.exp(s - m)
    p = p / jnp.sum(p, axis=-1, keepdims=True)
    pb = p.astype(jnp.bfloat16)
    o_ref[0] = jnp.dot(pb, v, preferred_element_type=jnp.float32).astype(jnp.bfloat16)


def _post_body(ao_ref, ow_ref, ob_ref, src_ref, g_ref, bt_ref, gw_ref, gb_ref,
               xn_ref, idx_ref, prb_ref, use_ref, aux_ref):
    i = pl.program_id(0)
    t = jnp.dot(ao_ref[...], ow_ref[...], preferred_element_type=jnp.float32)
    t = t + ob_ref[...] + src_ref[...]
    mean = jnp.mean(t, axis=-1, keepdims=True)
    var = jnp.mean((t - mean) ** 2, axis=-1, keepdims=True)
    xn = (t - mean) * lax.rsqrt(var + 1e-5) * g_ref[...] + bt_ref[...]
    xn_ref[...] = xn
    logits = jnp.dot(xn, gw_ref[...], preferred_element_type=jnp.float32)
    logits = logits + gb_ref[...]
    lane = lax.broadcasted_iota(jnp.int32, logits.shape, 1)
    m1 = jnp.max(logits, axis=-1, keepdims=True)
    i1 = jnp.min(jnp.where(logits == m1, lane, 2 ** 30), axis=-1, keepdims=True)
    masked = jnp.where(lane == i1, NEG, logits)
    m2 = jnp.max(masked, axis=-1, keepdims=True)
    i2 = jnp.min(jnp.where(masked == m2, lane, 2 ** 30), axis=-1, keepdims=True)
    p1 = 1.0 / (1.0 + jnp.exp(m2 - m1))
    p2 = 1.0 - p1
    idx_ref[...] = jnp.where(lane == 0, i1, jnp.where(lane == 1, i2, 0))
    prb_ref[...] = jnp.where(lane == 0, p1, jnp.where(lane == 1, p2, 0.0))
    ex = jnp.exp(logits - m1)
    gp = ex / jnp.sum(ex, axis=-1, keepdims=True)
    part = jnp.sum(gp, axis=0, keepdims=True)

    @pl.when(i == 0)
    def _():
        use_ref[...] = jnp.zeros_like(use_ref)

    use_ref[...] += part

    @pl.when(i == pl.num_programs(0) - 1)
    def _():
        u = use_ref[...] / float(S)
        aux_ref[...] = jnp.broadcast_to(float(E) * jnp.sum(u * u), aux_ref.shape)


def _expert_body(be_ref, xnb_ref, tok_ref, w1_ref, b1_ref, w2_ref, b2_ref,
                 sw_ref, y_ref):
    ids = tok_ref[...]
    sel = lax.broadcasted_iota(jnp.int32, (T, S), 1) == ids
    oh = sel.astype(jnp.bfloat16)
    xg = jnp.dot(oh, xnb_ref[...],
                 preferred_element_type=jnp.float32).astype(jnp.bfloat16)
    h = lax.dot_general(xg, w1_ref[0], (((1,), (1,)), ((), ())),
                        preferred_element_type=jnp.float32)
    h = h + b1_ref[0]
    h = 0.5 * h * (1.0 + lax.erf(h * np.float32(1.0 / np.sqrt(2.0))))
    hb = h.astype(jnp.bfloat16)
    y = lax.dot_general(hb, w2_ref[0], (((1,), (1,)), ((), ())),
                        preferred_element_type=jnp.float32)
    y = (y + b2_ref[0]) * sw_ref[...]
    y_ref[...] = y.astype(jnp.bfloat16)


def _combine_body(pos_ref, y_ref, xn_ref, g_ref, bt_ref, o_ref):
    p = pos_ref[...]
    i0 = p[:, 0:1]
    i1 = p[:, 1:2]
    io = lax.broadcasted_iota(jnp.int32, (TQ, PAD), 1)
    m = ((io == i0) | (io == i1)).astype(jnp.bfloat16)
    acc = jnp.dot(m, y_ref[...], preferred_element_type=jnp.float32)
    t = xn_ref[...] + acc
    mean = jnp.mean(t, axis=-1, keepdims=True)
    var = jnp.mean((t - mean) ** 2, axis=-1, keepdims=True)
    o_ref[...] = (t - mean) * lax.rsqrt(var + 1e-5) * g_ref[...] + bt_ref[...]


def kernel(src, in_proj_w, in_proj_b, out_w, out_b, norm1_g, norm1_b,
           gate_w, gate_b, W1, b1, W2, b2, norm2_g, norm2_b):
    src2 = src[0]
    srcb = src2.astype(jnp.bfloat16)

    wqkv = in_proj_w.T.astype(jnp.bfloat16)
    qkv = pl.pallas_call(
        _qkv_body,
        grid=(NTB,),
        in_specs=[pl.BlockSpec((TQ, D), lambda i: (i, 0)),
                  pl.BlockSpec((D, 3 * D), lambda i: (0, 0)),
                  pl.BlockSpec((1, 3 * D), lambda i: (0, 0))],
        out_specs=pl.BlockSpec((TQ, 3 * D), lambda i: (i, 0)),
        out_shape=jax.ShapeDtypeStruct((S, 3 * D), jnp.bfloat16),
    )(srcb, wqkv, in_proj_b[None, :])

    def heads(t):
        return t.reshape(S, H, DH).transpose(1, 0, 2)

    qh = heads(qkv[:, :D])
    kh = heads(qkv[:, D:2 * D])
    vh = heads(qkv[:, 2 * D:])

    ao = pl.pallas_call(
        _attn_body,
        grid=(H, NTB),
        in_specs=[pl.BlockSpec((1, TQ, DH), lambda h, i: (h, i, 0)),
                  pl.BlockSpec((1, S, DH), lambda h, i: (h, 0, 0)),
                  pl.BlockSpec((1, S, DH), lambda h, i: (h, 0, 0))],
        out_specs=pl.BlockSpec((1, TQ, DH), lambda h, i: (h, i, 0)),
        out_shape=jax.ShapeDtypeStruct((H, S, DH), jnp.bfloat16),
    )(qh, kh, vh)
    aoc = ao.transpose(1, 0, 2).reshape(S, D)

    owt = out_w.T.astype(jnp.bfloat16)
    gwp = jnp.zeros((D, EPAD), jnp.float32).at[:, :E].set(gate_w.T)
    gbp = jnp.full((1, EPAD), NEG, jnp.float32).at[0, :E].set(gate_b)
    xn, idx_o, prb_o, use_o, aux_o = pl.pallas_call(
        _post_body,
        grid=(NTB,),
        in_specs=[pl.BlockSpec((TQ, D), lambda i: (i, 0)),
                  pl.BlockSpec((D, D), lambda i: (0, 0)),
                  pl.BlockSpec((1, D), lambda i: (0, 0)),
                  pl.BlockSpec((TQ, D), lambda i: (i, 0)),
                  pl.BlockSpec((1, D), lambda i: (0, 0)),
                  pl.BlockSpec((1, D), lambda i: (0, 0)),
                  pl.BlockSpec((D, EPAD), lambda i: (0, 0)),
                  pl.BlockSpec((1, EPAD), lambda i: (0, 0))],
        out_specs=[pl.BlockSpec((TQ, D), lambda i: (i, 0)),
                   pl.BlockSpec((TQ, EPAD), lambda i: (i, 0)),
                   pl.BlockSpec((TQ, EPAD), lambda i: (i, 0)),
                   pl.BlockSpec((1, EPAD), lambda i: (0, 0)),
                   pl.BlockSpec((1, EPAD), lambda i: (0, 0))],
        out_shape=[jax.ShapeDtypeStruct((S, D), jnp.float32),
                   jax.ShapeDtypeStruct((S, EPAD), jnp.int32),
                   jax.ShapeDtypeStruct((S, EPAD), jnp.float32),
                   jax.ShapeDtypeStruct((1, EPAD), jnp.float32),
                   jax.ShapeDtypeStruct((1, EPAD), jnp.float32)],
    )(aoc, owt, out_b[None, :], src2, norm1_g[None, :], norm1_b[None, :],
      gwp, gbp)

    flat_e = idx_o[:, :K].reshape(-1)
    flat_w = prb_o[:, :K].reshape(-1)
    oh = (flat_e[:, None] == jnp.arange(E)[None, :]).astype(jnp.int32)
    ranks = jnp.cumsum(oh, axis=0) - oh
    rank = jnp.take_along_axis(ranks, flat_e[:, None], axis=1)[:, 0]
    counts = jnp.sum(oh, axis=0)
    padded = ((counts + T - 1) // T) * T
    ends = jnp.cumsum(padded)
    offs = ends - padded
    pos = (offs[flat_e] + rank).astype(jnp.int32)
    slot_tok = jnp.zeros((PAD,), jnp.int32).at[pos].set(
        jnp.arange(S * K, dtype=jnp.int32) // K)
    slot_w = jnp.zeros((PAD,), jnp.float32).at[pos].set(flat_w)
    blk_e = jnp.minimum(
        jnp.searchsorted(ends, jnp.arange(NBLK, dtype=jnp.int32) * T,
                         side='right'),
        E - 1).astype(jnp.int32)

    xnb = xn.astype(jnp.bfloat16)
    w1b = W1.astype(jnp.bfloat16)
    w2b = W2.astype(jnp.bfloat16)
    grid_spec = pltpu.PrefetchScalarGridSpec(
        num_scalar_prefetch=1,
        grid=(NBLK,),
        in_specs=[pl.BlockSpec((S, D), lambda b, be: (0, 0)),
                  pl.BlockSpec((T, 1), lambda b, be: (b, 0)),
                  pl.BlockSpec((1, DFF, D), lambda b, be: (be[b], 0, 0)),
                  pl.BlockSpec((1, 1, DFF), lambda b, be: (be[b], 0, 0)),
                  pl.BlockSpec((1, D, DFF), lambda b, be: (be[b], 0, 0)),
                  pl.BlockSpec((1, 1, D), lambda b, be: (be[b], 0, 0)),
                  pl.BlockSpec((T, 1), lambda b, be: (b, 0))],
        out_specs=pl.BlockSpec((T, D), lambda b, be: (b, 0)),
    )
    y = pl.pallas_call(
        _expert_body,
        grid_spec=grid_spec,
        out_shape=jax.ShapeDtypeStruct((PAD, D), jnp.bfloat16),
    )(blk_e, xnb, slot_tok[:, None], w1b, b1[:, None, :], w2b,
      b2[:, None, :], slot_w[:, None])

    out = pl.pallas_call(
        _combine_body,
        grid=(NTB,),
        in_specs=[pl.BlockSpec((TQ, K), lambda i: (i, 0)),
                  pl.BlockSpec((PAD, D), lambda i: (0, 0)),
                  pl.BlockSpec((TQ, D), lambda i: (i, 0)),
                  pl.BlockSpec((1, D), lambda i: (0, 0)),
                  pl.BlockSpec((1, D), lambda i: (0, 0))],
        out_specs=pl.BlockSpec((TQ, D), lambda i: (i, 0)),
        out_shape=jax.ShapeDtypeStruct((S, D), jnp.float32),
    )(pos.reshape(S, K), y, xn, norm2_g[None, :], norm2_b[None, :])

    return out[None], aux_o[0, 0]


# speedup vs baseline: 1.8369x; 1.2184x over previous
import functools

import jax
import jax.numpy as jnp
import numpy as np
from jax import lax
from jax.experimental import pallas as pl
from jax.experimental.pallas import tpu as pltpu

S, D, H, DFF, E, K = 2048, 768, 12, 3072, 8, 2
DH = D // H
TQ = 256
NTB = S // TQ
T = 128
PAD = S * K + E * T
NBLK = PAD // T
EPAD = 128
NEG = -1e30


def _qkv_body(x_ref, w_ref, b_ref, o_ref):
    acc = jnp.dot(x_ref[...], w_ref[...], preferred_element_type=jnp.float32)
    o_ref[...] = (acc + b_ref[...]).astype(jnp.bfloat16)


def _attn_body(q_ref, k_ref, v_ref, o_ref):
    q = q_ref[...]
    k = k_ref[...]
    v = v_ref[...]
    lane = lax.broadcasted_iota(jnp.int32, (1, 2 * DH), 1)
    scale = np.float32(1.0 / np.sqrt(DH))
    acc = None
    for hh in range(2):
        msk = (lane >= hh * DH) & (lane < (hh + 1) * DH)
        qm = jnp.where(msk, q, jnp.bfloat16(0))
        km = jnp.where(msk, k, jnp.bfloat16(0))
        vm = jnp.where(msk, v, jnp.bfloat16(0))
        s = lax.dot_general(qm, km, (((1,), (1,)), ((), ())),
                            preferred_element_type=jnp.float32) * scale
        m = jnp.max(s, axis=-1, keepdims=True)
        p = jnp.exp(s - m)
        p = p / jnp.sum(p, axis=-1, keepdims=True)
        o = jnp.dot(p.astype(jnp.bfloat16), vm,
                    preferred_element_type=jnp.float32)
        acc = o if acc is None else acc + o
    o_ref[...] = acc.astype(jnp.bfloat16)


def _post_body(ao_ref, ow_ref, ob_ref, src_ref, g_ref, bt_ref, gw_ref, gb_ref,
               xn_ref, idx_ref, prb_ref, use_ref, aux_ref):
    i = pl.program_id(0)
    t = jnp.dot(ao_ref[...], ow_ref[...], preferred_element_type=jnp.float32)
    t = t + ob_ref[...] + src_ref[...]
    mean = jnp.mean(t, axis=-1, keepdims=True)
    var = jnp.mean((t - mean) ** 2, axis=-1, keepdims=True)
    xn = (t - mean) * lax.rsqrt(var + 1e-5) * g_ref[...] + bt_ref[...]
    xn_ref[...] = xn
    logits = jnp.dot(xn, gw_ref[...], preferred_element_type=jnp.float32)
    logits = logits + gb_ref[...]
    lane = lax.broadcasted_iota(jnp.int32, logits.shape, 1)
    m1 = jnp.max(logits, axis=-1, keepdims=True)
    i1 = jnp.min(jnp.where(logits == m1, lane, 2 ** 30), axis=-1, keepdims=True)
    masked = jnp.where(lane == i1, NEG, logits)
    m2 = jnp.max(masked, axis=-1, keepdims=True)
    i2 = jnp.min(jnp.where(masked == m2, lane, 2 ** 30), axis=-1, keepdims=True)
    p1 = 1.0 / (1.0 + jnp.exp(m2 - m1))
    p2 = 1.0 - p1
    idx_ref[...] = jnp.where(lane == 0, i1, jnp.where(lane == 1, i2, 0))
    prb_ref[...] = jnp.where(lane == 0, p1, jnp.where(lane == 1, p2, 0.0))
    ex = jnp.exp(logits - m1)
    gp = ex / jnp.sum(ex, axis=-1, keepdims=True)
    part = jnp.sum(gp, axis=0, keepdims=True)

    @pl.when(i == 0)
    def _():
        use_ref[...] = jnp.zeros_like(use_ref)

    use_ref[...] += part

    @pl.when(i == pl.num_programs(0) - 1)
    def _():
        u = use_ref[...] / float(S)
        aux_ref[...] = jnp.broadcast_to(float(E) * jnp.sum(u * u), aux_ref.shape)


def _expert_body(be_ref, xnb_ref, tok_ref, w1_ref, b1_ref, w2_ref, b2_ref,
                 sw_ref, y_ref):
    ids = tok_ref[...]
    sel = lax.broadcasted_iota(jnp.int32, (T, S), 1) == ids
    oh = sel.astype(jnp.bfloat16)
    xg = jnp.dot(oh, xnb_ref[...],
                 preferred_element_type=jnp.float32).astype(jnp.bfloat16)
    h = lax.dot_general(xg, w1_ref[0], (((1,), (1,)), ((), ())),
                        preferred_element_type=jnp.float32)
    h = h + b1_ref[0]
    h = 0.5 * h * (1.0 + lax.erf(h * np.float32(1.0 / np.sqrt(2.0))))
    hb = h.astype(jnp.bfloat16)
    y = lax.dot_general(hb, w2_ref[0], (((1,), (1,)), ((), ())),
                        preferred_element_type=jnp.float32)
    y = (y + b2_ref[0]) * sw_ref[...]
    y_ref[...] = y.astype(jnp.bfloat16)


def _combine_body(pos_ref, y_ref, xn_ref, g_ref, bt_ref, o_ref):
    p = pos_ref[...]
    i0 = p[:, 0:1]
    i1 = p[:, 1:2]
    io = lax.broadcasted_iota(jnp.int32, (TQ, PAD), 1)
    m = ((io == i0) | (io == i1)).astype(jnp.bfloat16)
    acc = jnp.dot(m, y_ref[...], preferred_element_type=jnp.float32)
    t = xn_ref[...] + acc
    mean = jnp.mean(t, axis=-1, keepdims=True)
    var = jnp.mean((t - mean) ** 2, axis=-1, keepdims=True)
    o_ref[...] = (t - mean) * lax.rsqrt(var + 1e-5) * g_ref[...] + bt_ref[...]


def kernel(src, in_proj_w, in_proj_b, out_w, out_b, norm1_g, norm1_b,
           gate_w, gate_b, W1, b1, W2, b2, norm2_g, norm2_b):
    src2 = src[0]
    srcb = src2.astype(jnp.bfloat16)

    wqkv = in_proj_w.T.astype(jnp.bfloat16)
    qkv = pl.pallas_call(
        _qkv_body,
        grid=(NTB,),
        in_specs=[pl.BlockSpec((TQ, D), lambda i: (i, 0)),
                  pl.BlockSpec((D, 3 * D), lambda i: (0, 0)),
                  pl.BlockSpec((1, 3 * D), lambda i: (0, 0))],
        out_specs=pl.BlockSpec((TQ, 3 * D), lambda i: (i, 0)),
        out_shape=jax.ShapeDtypeStruct((S, 3 * D), jnp.bfloat16),
    )(srcb, wqkv, in_proj_b[None, :])

    PH = H // 2
    aoc = pl.pallas_call(
        _attn_body,
        grid=(PH, NTB),
        in_specs=[pl.BlockSpec((TQ, 2 * DH), lambda p, i: (i, p)),
                  pl.BlockSpec((S, 2 * DH), lambda p, i: (0, PH + p)),
                  pl.BlockSpec((S, 2 * DH), lambda p, i: (0, 2 * PH + p))],
        out_specs=pl.BlockSpec((TQ, 2 * DH), lambda p, i: (i, p)),
        out_shape=jax.ShapeDtypeStruct((S, D), jnp.bfloat16),
    )(qkv, qkv, qkv)

    owt = out_w.T.astype(jnp.bfloat16)
    gwp = jnp.zeros((D, EPAD), jnp.float32).at[:, :E].set(gate_w.T)
    gbp = jnp.full((1, EPAD), NEG, jnp.float32).at[0, :E].set(gate_b)
    xn, idx_o, prb_o, use_o, aux_o = pl.pallas_call(
        _post_body,
        grid=(NTB,),
        in_specs=[pl.BlockSpec((TQ, D), lambda i: (i, 0)),
                  pl.BlockSpec((D, D), lambda i: (0, 0)),
                  pl.BlockSpec((1, D), lambda i: (0, 0)),
                  pl.BlockSpec((TQ, D), lambda i: (i, 0)),
                  pl.BlockSpec((1, D), lambda i: (0, 0)),
                  pl.BlockSpec((1, D), lambda i: (0, 0)),
                  pl.BlockSpec((D, EPAD), lambda i: (0, 0)),
                  pl.BlockSpec((1, EPAD), lambda i: (0, 0))],
        out_specs=[pl.BlockSpec((TQ, D), lambda i: (i, 0)),
                   pl.BlockSpec((TQ, EPAD), lambda i: (i, 0)),
                   pl.BlockSpec((TQ, EPAD), lambda i: (i, 0)),
                   pl.BlockSpec((1, EPAD), lambda i: (0, 0)),
                   pl.BlockSpec((1, EPAD), lambda i: (0, 0))],
        out_shape=[jax.ShapeDtypeStruct((S, D), jnp.float32),
                   jax.ShapeDtypeStruct((S, EPAD), jnp.int32),
                   jax.ShapeDtypeStruct((S, EPAD), jnp.float32),
                   jax.ShapeDtypeStruct((1, EPAD), jnp.float32),
                   jax.ShapeDtypeStruct((1, EPAD), jnp.float32)],
    )(aoc, owt, out_b[None, :], src2, norm1_g[None, :], norm1_b[None, :],
      gwp, gbp)

    flat_e = idx_o[:, :K].reshape(-1)
    flat_w = prb_o[:, :K].reshape(-1)
    oh = (flat_e[:, None] == jnp.arange(E)[None, :]).astype(jnp.int32)
    ranks = jnp.cumsum(oh, axis=0) - oh
    rank = jnp.take_along_axis(ranks, flat_e[:, None], axis=1)[:, 0]
    counts = jnp.sum(oh, axis=0)
    padded = ((counts + T - 1) // T) * T
    ends = jnp.cumsum(padded)
    offs = ends - padded
    pos = (offs[flat_e] + rank).astype(jnp.int32)
    slot_tok = jnp.zeros((PAD,), jnp.int32).at[pos].set(
        jnp.arange(S * K, dtype=jnp.int32) // K)
    slot_w = jnp.zeros((PAD,), jnp.float32).at[pos].set(flat_w)
    blk_e = jnp.minimum(
        jnp.searchsorted(ends, jnp.arange(NBLK, dtype=jnp.int32) * T,
                         side='right'),
        E - 1).astype(jnp.int32)

    xnb = xn.astype(jnp.bfloat16)
    w1b = W1.astype(jnp.bfloat16)
    w2b = W2.astype(jnp.bfloat16)
    grid_spec = pltpu.PrefetchScalarGridSpec(
        num_scalar_prefetch=1,
        grid=(NBLK,),
        in_specs=[pl.BlockSpec((S, D), lambda b, be: (0, 0)),
                  pl.BlockSpec((T, 1), lambda b, be: (b, 0)),
                  pl.BlockSpec((1, DFF, D), lambda b, be: (be[b], 0, 0)),
                  pl.BlockSpec((1, 1, DFF), lambda b, be: (be[b], 0, 0)),
                  pl.BlockSpec((1, D, DFF), lambda b, be: (be[b], 0, 0)),
                  pl.BlockSpec((1, 1, D), lambda b, be: (be[b], 0, 0)),
                  pl.BlockSpec((T, 1), lambda b, be: (b, 0))],
        out_specs=pl.BlockSpec((T, D), lambda b, be: (b, 0)),
    )
    y = pl.pallas_call(
        _expert_body,
        grid_spec=grid_spec,
        out_shape=jax.ShapeDtypeStruct((PAD, D), jnp.bfloat16),
    )(blk_e, xnb, slot_tok[:, None], w1b, b1[:, None, :], w2b,
      b2[:, None, :], slot_w[:, None])

    out = pl.pallas_call(
        _combine_body,
        grid=(NTB,),
        in_specs=[pl.BlockSpec((TQ, K), lambda i: (i, 0)),
                  pl.BlockSpec((PAD, D), lambda i: (0, 0)),
                  pl.BlockSpec((TQ, D), lambda i: (i, 0)),
                  pl.BlockSpec((1, D), lambda i: (0, 0)),
                  pl.BlockSpec((1, D), lambda i: (0, 0))],
        out_specs=pl.BlockSpec((TQ, D), lambda i: (i, 0)),
        out_shape=jax.ShapeDtypeStruct((S, D), jnp.float32),
    )(pos.reshape(S, K), y, xn, norm2_g[None, :], norm2_b[None, :])

    return out[None], aux_o[0, 0]
